# baseline (speedup 1.0000x reference)
import functools

import jax
import jax.numpy as jnp
from jax import lax
from jax.experimental import pallas as pl
from jax.experimental.pallas import tpu as pltpu
from jax.experimental.pallas import tpu_sc as plsc

N = 10000
E = 320000
D = 128

NC = 2
NS = 16
NW = NC * NS
CH = 80
NCH = E // (NW * CH)
ROWS_PT = 624
TAIL = N - NS * ROWS_PT
ZCHUNKS = [(k * CH, CH) for k in range(7)] + [(7 * CH, 64)]
LANES = 16
NJ = D // LANES


def _mm_body(h_ref, w_ref, o_ref):
    w = w_ref[0] + w_ref[1]
    o_ref[...] = jnp.dot(h_ref[...], w, preferred_element_type=jnp.float32)


def _combine_body(p_ref, b_ref, o_ref):
    o_ref[...] = p_ref[0] + p_ref[1] + b_ref[...]


def _sinv_body(ef_ref, o_ref):
    o_ref[...] = 1.0 / jnp.sum(jnp.exp(ef_ref[...]), axis=1)


_GATHER_DNUMS = lax.GatherDimensionNumbers(
    offset_dims=(), collapsed_slice_dims=(0,), start_index_map=(0,))


def _lane_shuffle(x, idx):
    return lax.gather(x, idx[:, None], _GATHER_DNUMS, (1,),
                      mode=lax.GatherScatterMode.PROMISE_IN_BOUNDS)


def _sc_edge_pass(hs_hbm, ei_hbm, ef_hbm, siv_hbm, out_hbm,
                  src_v, dst_v, ef_v, rows_v, siv_v, acc,
                  sem_i, sem_e, sem_g, sem_s):
    cid = lax.axis_index("c")
    sid = lax.axis_index("s")
    wid = cid * NS + sid

    def zrow(r, carry):
        for j in range(NJ):
            rows_v[0, r, pl.ds(LANES * j, LANES)] = jnp.zeros(
                (LANES,), jnp.float32)
        return carry

    lax.fori_loop(0, CH, zrow, 0)
    base0 = sid * ROWS_PT
    for off, sz in ZCHUNKS:
        pltpu.sync_copy(rows_v.at[0, pl.ds(0, sz)],
                        acc.at[pl.ds(base0 + off, sz)])

    @pl.when(sid == NS - 1)
    def _zero_tail():
        pltpu.sync_copy(rows_v.at[0, pl.ds(0, TAIL)],
                        acc.at[pl.ds(NS * ROWS_PT, TAIL)])

    plsc.subcore_barrier()

    def _chunk_base(i):
        return (wid + i * NW) * CH

    def _idx_start(i, s):
        base = _chunk_base(i)
        pltpu.async_copy(ei_hbm.at[pl.ds(base, CH)], src_v.at[s], sem_i)
        pltpu.async_copy(ei_hbm.at[pl.ds(E + base, CH)], dst_v.at[s], sem_i)

    def _idx_wait(i, s):
        base = _chunk_base(i)
        pltpu.make_async_copy(
            ei_hbm.at[pl.ds(base, CH)], src_v.at[s], sem_i).wait()
        pltpu.make_async_copy(
            ei_hbm.at[pl.ds(E + base, CH)], dst_v.at[s], sem_i).wait()

    def _ef_start(i, b):
        base = _chunk_base(i)
        pltpu.async_copy(ef_hbm.at[pl.ds(base, CH)], ef_v.at[b], sem_e)
        pltpu.async_copy(siv_hbm.at[pl.ds(base, CH)], siv_v.at[b], sem_e)

    def _ef_wait(i, b):
        base = _chunk_base(i)
        pltpu.make_async_copy(
            ef_hbm.at[pl.ds(base, CH)], ef_v.at[b], sem_e).wait()
        pltpu.make_async_copy(
            siv_hbm.at[pl.ds(base, CH)], siv_v.at[b], sem_e).wait()

    _idx_start(0, 0)
    _ef_start(0, 0)
    _idx_wait(0, 0)
    pltpu.async_copy(hs_hbm.at[src_v.at[0]], rows_v.at[0], sem_g)
    _idx_start(1, 1)

    def chunk_body(i, carry):
        b = jnp.bitwise_and(i, 1)
        nb = 1 - b
        s0 = lax.rem(i, 3)
        s1 = lax.rem(i + 1, 3)
        s2 = lax.rem(i + 2, 3)
        sp = lax.rem(i + 2, 3)
        pltpu.make_async_copy(
            hs_hbm.at[src_v.at[s0]], rows_v.at[b], sem_g).wait()
        _ef_wait(i, b)

        @pl.when(i < NCH - 1)
        def _start_next_gather():
            _idx_wait(i + 1, s1)
            pltpu.async_copy(hs_hbm.at[src_v.at[s1]], rows_v.at[nb], sem_g)

            @pl.when(i > 0)
            def _drain_prev_scatter():
                pltpu.make_async_copy(
                    ef_v.at[nb], acc.at[dst_v.at[sp]], sem_s).wait()

            _ef_start(i + 1, nb)

        @pl.when(i < NCH - 2)
        def _start_next_idx():
            _idx_start(i + 2, s2)

        @plsc.parallel_loop(0, CH, unroll=2)
        def row(r):
            g16 = pl.multiple_of((r >> 4) << 4, 8)
            gv = siv_v[b, pl.ds(g16, LANES)]
            lid = jnp.broadcast_to(jnp.bitwise_and(r, 15), (LANES,))
            inv = _lane_shuffle(gv, lid)
            for j in range(NJ):
                x = jnp.exp(ef_v[b, r, pl.ds(LANES * j, LANES)])
                ef_v[b, r, pl.ds(LANES * j, LANES)] = (
                    x * (rows_v[b, r, pl.ds(LANES * j, LANES)] * inv))

        pltpu.async_copy(ef_v.at[b], acc.at[dst_v.at[s0]], sem_s, add=True)
        return carry

    lax.fori_loop(0, NCH, chunk_body, 0)

    lastb = jnp.int32((NCH - 1) % 2)
    pltpu.make_async_copy(
        ef_v.at[1 - lastb], acc.at[dst_v.at[jnp.int32((NCH - 2) % 3)]],
        sem_s).wait()
    pltpu.make_async_copy(
        ef_v.at[lastb], acc.at[dst_v.at[jnp.int32((NCH - 1) % 3)]],
        sem_s).wait()
    plsc.subcore_barrier()

    for off, sz in ZCHUNKS:
        start = base0 + off
        pltpu.sync_copy(acc.at[pl.ds(start, sz)], rows_v.at[0, pl.ds(0, sz)])
        pltpu.sync_copy(rows_v.at[0, pl.ds(0, sz)],
                        out_hbm.at[cid, pl.ds(start, sz)])

    @pl.when(sid == NS - 1)
    def _copy_tail():
        pltpu.sync_copy(acc.at[pl.ds(NS * ROWS_PT, TAIL)],
                        rows_v.at[0, pl.ds(0, TAIL)])
        pltpu.sync_copy(rows_v.at[0, pl.ds(0, TAIL)],
                        out_hbm.at[cid, pl.ds(NS * ROWS_PT, TAIL)])


_sc_call = functools.partial(
    pl.kernel,
    mesh=plsc.VectorSubcoreMesh(core_axis_name="c", subcore_axis_name="s"),
    out_type=jax.ShapeDtypeStruct((NC, N, D), jnp.float32),
    scratch_types=[
        pltpu.VMEM((3, CH), jnp.int32),
        pltpu.VMEM((3, CH), jnp.int32),
        pltpu.VMEM((2, CH, D), jnp.float32),
        pltpu.VMEM((2, CH, D), jnp.float32),
        pltpu.VMEM((2, CH), jnp.float32),
        pltpu.VMEM_SHARED((N, D), jnp.float32),
        pltpu.SemaphoreType.DMA,
        pltpu.SemaphoreType.DMA,
        pltpu.SemaphoreType.DMA,
        pltpu.SemaphoreType.DMA,
    ],
)(_sc_edge_pass)


def kernel(h, edge_index, edge_feat, weight, bias):
    hs = pl.pallas_call(
        _mm_body,
        grid=(5,),
        in_specs=[
            pl.BlockSpec((2000, D), lambda i: (i, 0)),
            pl.BlockSpec((2, D, D), lambda i: (0, 0, 0)),
        ],
        out_specs=pl.BlockSpec((2000, D), lambda i: (i, 0)),
        out_shape=jax.ShapeDtypeStruct((N, D), jnp.float32),
    )(h, weight)

    sinv = pl.pallas_call(
        _sinv_body,
        grid=(625,),
        in_specs=[pl.BlockSpec((512, D), lambda i: (i, 0))],
        out_specs=pl.BlockSpec((512,), lambda i: (i,)),
        out_shape=jax.ShapeDtypeStruct((E,), jnp.float32),
    )(edge_feat)

    partials = _sc_call(hs, edge_index.reshape(2 * E), edge_feat, sinv)

    out = pl.pallas_call(
        _combine_body,
        grid=(5,),
        in_specs=[
            pl.BlockSpec((NC, 2000, D), lambda i: (0, i, 0)),
            pl.BlockSpec((1, D), lambda i: (0, 0)),
        ],
        out_specs=pl.BlockSpec((2000, D), lambda i: (i, 0)),
        out_shape=jax.ShapeDtypeStruct((N, D), jnp.float32),
    )(partials, bias.reshape(1, D))
    return out


# speedup vs baseline: 2.4885x; 2.4885x over previous
import functools

import jax
import jax.numpy as jnp
from jax import lax
from jax.experimental import pallas as pl
from jax.experimental.pallas import tpu as pltpu
from jax.experimental.pallas import tpu_sc as plsc

N = 10000
E = 320000
D = 128

NC = 2
NS = 16
NW = NC * NS
CH = 80
NCH = E // (NW * CH)
ROWS_PT = 624
TAIL = N - NS * ROWS_PT
ZCHUNKS = [(k * CH, CH) for k in range(7)] + [(7 * CH, 64)]
LANES = 16
NJ = D // LANES


def _mm_body(h_ref, w_ref, o_ref):
    w = w_ref[0] + w_ref[1]
    o_ref[...] = jnp.dot(h_ref[...], w, preferred_element_type=jnp.float32)


def _combine_body(p_ref, b_ref, o_ref):
    o_ref[...] = p_ref[0] + p_ref[1] + b_ref[...]


_GATHER_DNUMS = lax.GatherDimensionNumbers(
    offset_dims=(), collapsed_slice_dims=(0,), start_index_map=(0,))


def _lane_shuffle(x, idx):
    return lax.gather(x, idx[:, None], _GATHER_DNUMS, (1,),
                      mode=lax.GatherScatterMode.PROMISE_IN_BOUNDS)


def _sc_edge_pass(hs_hbm, ei_hbm, ef_hbm, out_hbm,
                  src_v, dst_v, ef_v, rows_v, invb_v, acc,
                  sem_i, sem_e, sem_g, sem_s):
    cid = lax.axis_index("c")
    sid = lax.axis_index("s")
    wid = cid * NS + sid

    def zrow(r, carry):
        for j in range(NJ):
            rows_v[0, r, pl.ds(LANES * j, LANES)] = jnp.zeros(
                (LANES,), jnp.float32)
        return carry

    lax.fori_loop(0, CH, zrow, 0)
    base0 = sid * ROWS_PT
    for off, sz in ZCHUNKS:
        pltpu.sync_copy(rows_v.at[0, pl.ds(0, sz)],
                        acc.at[pl.ds(base0 + off, sz)])

    @pl.when(sid == NS - 1)
    def _zero_tail():
        pltpu.sync_copy(rows_v.at[0, pl.ds(0, TAIL)],
                        acc.at[pl.ds(NS * ROWS_PT, TAIL)])

    plsc.subcore_barrier()

    lane = lax.iota(jnp.int32, LANES)
    bfly = [lane ^ (1 << k) for k in range(4)]

    def _chunk_base(i):
        return (wid + i * NW) * CH

    def _idx_start(i, s):
        base = _chunk_base(i)
        pltpu.async_copy(ei_hbm.at[pl.ds(base, CH)], src_v.at[s], sem_i)
        pltpu.async_copy(ei_hbm.at[pl.ds(E + base, CH)], dst_v.at[s], sem_i)

    def _idx_wait(i, s):
        base = _chunk_base(i)
        pltpu.make_async_copy(
            ei_hbm.at[pl.ds(base, CH)], src_v.at[s], sem_i).wait()
        pltpu.make_async_copy(
            ei_hbm.at[pl.ds(E + base, CH)], dst_v.at[s], sem_i).wait()

    _idx_start(0, 0)
    pltpu.async_copy(ef_hbm.at[pl.ds(_chunk_base(0), CH)], ef_v.at[0], sem_e)
    _idx_wait(0, 0)
    pltpu.async_copy(hs_hbm.at[src_v.at[0]], rows_v.at[0], sem_g)
    _idx_start(1, 1)

    def chunk_body(i, carry):
        b = jnp.bitwise_and(i, 1)
        nb = 1 - b
        s0 = lax.rem(i, 3)
        s1 = lax.rem(i + 1, 3)
        s2 = lax.rem(i + 2, 3)
        sp = lax.rem(i + 2, 3)
        pltpu.make_async_copy(
            hs_hbm.at[src_v.at[s0]], rows_v.at[b], sem_g).wait()
        pltpu.make_async_copy(
            ef_hbm.at[pl.ds(_chunk_base(i), CH)], ef_v.at[b], sem_e).wait()

        @pl.when(i < NCH - 1)
        def _start_next_gather():
            _idx_wait(i + 1, s1)
            pltpu.async_copy(hs_hbm.at[src_v.at[s1]], rows_v.at[nb], sem_g)

            @pl.when(i > 0)
            def _drain_prev_scatter():
                pltpu.make_async_copy(
                    ef_v.at[nb], acc.at[dst_v.at[sp]], sem_s).wait()

            pltpu.async_copy(
                ef_hbm.at[pl.ds(_chunk_base(i + 1), CH)], ef_v.at[nb], sem_e)

        @pl.when(i < NCH - 2)
        def _start_next_idx():
            _idx_start(i + 2, s2)

        @plsc.parallel_loop(0, CH, unroll=2)
        def rowsum(r):
            xs = []
            for j in range(NJ):
                x = jnp.exp(ef_v[b, r, pl.ds(LANES * j, LANES)])
                ef_v[b, r, pl.ds(LANES * j, LANES)] = x
                xs.append(x)
            t = [xs[2 * j] + xs[2 * j + 1] for j in range(4)]
            u = [t[0] + t[1], t[2] + t[3]]
            s = u[0] + u[1]
            for p in bfly:
                s = s + _lane_shuffle(s, p)
            invb_v[b, pl.ds(pl.multiple_of(r * LANES, 8), LANES)] = 1.0 / s

        @plsc.parallel_loop(0, CH, unroll=2)
        def rowmul(r):
            inv = invb_v[b, pl.ds(pl.multiple_of(r * LANES, 8), LANES)]
            for j in range(NJ):
                ef_v[b, r, pl.ds(LANES * j, LANES)] = (
                    ef_v[b, r, pl.ds(LANES * j, LANES)]
                    * (rows_v[b, r, pl.ds(LANES * j, LANES)] * inv))

        pltpu.async_copy(ef_v.at[b], acc.at[dst_v.at[s0]], sem_s, add=True)
        return carry

    lax.fori_loop(0, NCH, chunk_body, 0)

    lastb = jnp.int32((NCH - 1) % 2)
    pltpu.make_async_copy(
        ef_v.at[1 - lastb], acc.at[dst_v.at[jnp.int32((NCH - 2) % 3)]],
        sem_s).wait()
    pltpu.make_async_copy(
        ef_v.at[lastb], acc.at[dst_v.at[jnp.int32((NCH - 1) % 3)]],
        sem_s).wait()
    plsc.subcore_barrier()

    for off, sz in ZCHUNKS:
        start = base0 + off
        pltpu.sync_copy(acc.at[pl.ds(start, sz)], rows_v.at[0, pl.ds(0, sz)])
        pltpu.sync_copy(rows_v.at[0, pl.ds(0, sz)],
                        out_hbm.at[cid, pl.ds(start, sz)])

    @pl.when(sid == NS - 1)
    def _copy_tail():
        pltpu.sync_copy(acc.at[pl.ds(NS * ROWS_PT, TAIL)],
                        rows_v.at[0, pl.ds(0, TAIL)])
        pltpu.sync_copy(rows_v.at[0, pl.ds(0, TAIL)],
                        out_hbm.at[cid, pl.ds(NS * ROWS_PT, TAIL)])


_sc_call = functools.partial(
    pl.kernel,
    mesh=plsc.VectorSubcoreMesh(core_axis_name="c", subcore_axis_name="s"),
    out_type=jax.ShapeDtypeStruct((NC, N, D), jnp.float32),
    scratch_types=[
        pltpu.VMEM((3, CH), jnp.int32),
        pltpu.VMEM((3, CH), jnp.int32),
        pltpu.VMEM((2, CH, D), jnp.float32),
        pltpu.VMEM((2, CH, D), jnp.float32),
        pltpu.VMEM((2, CH * LANES), jnp.float32),
        pltpu.VMEM_SHARED((N, D), jnp.float32),
        pltpu.SemaphoreType.DMA,
        pltpu.SemaphoreType.DMA,
        pltpu.SemaphoreType.DMA,
        pltpu.SemaphoreType.DMA,
    ],
)(_sc_edge_pass)


def kernel(h, edge_index, edge_feat, weight, bias):
    hs = pl.pallas_call(
        _mm_body,
        grid=(5,),
        in_specs=[
            pl.BlockSpec((2000, D), lambda i: (i, 0)),
            pl.BlockSpec((2, D, D), lambda i: (0, 0, 0)),
        ],
        out_specs=pl.BlockSpec((2000, D), lambda i: (i, 0)),
        out_shape=jax.ShapeDtypeStruct((N, D), jnp.float32),
    )(h, weight)

    partials = _sc_call(hs, edge_index.reshape(2 * E), edge_feat)

    out = pl.pallas_call(
        _combine_body,
        grid=(5,),
        in_specs=[
            pl.BlockSpec((NC, 2000, D), lambda i: (0, i, 0)),
            pl.BlockSpec((1, D), lambda i: (0, 0)),
        ],
        out_specs=pl.BlockSpec((2000, D), lambda i: (i, 0)),
        out_shape=jax.ShapeDtypeStruct((N, D), jnp.float32),
    )(partials, bias.reshape(1, D))
    return out


# speedup vs baseline: 2.5030x; 1.0058x over previous
import functools

import jax
import jax.numpy as jnp
from jax import lax
from jax.experimental import pallas as pl
from jax.experimental.pallas import tpu as pltpu
from jax.experimental.pallas import tpu_sc as plsc

N = 10000
E = 320000
D = 128

NC = 2
NS = 16
NW = NC * NS
CH = 80
NCH = E // (NW * CH)
ROWS_PT = 624
TAIL = N - NS * ROWS_PT
ZCHUNKS = [(k * CH, CH) for k in range(7)] + [(7 * CH, 64)]
LANES = 16
NJ = D // LANES


def _mm_body(h_ref, w_ref, o_ref):
    w = w_ref[0] + w_ref[1]
    o_ref[...] = jnp.dot(h_ref[...], w, preferred_element_type=jnp.float32)


def _combine_body(p_ref, b_ref, o_ref):
    o_ref[...] = p_ref[0] + p_ref[1] + b_ref[...]


_GATHER_DNUMS = lax.GatherDimensionNumbers(
    offset_dims=(), collapsed_slice_dims=(0,), start_index_map=(0,))


def _lane_shuffle(x, idx):
    return lax.gather(x, idx[:, None], _GATHER_DNUMS, (1,),
                      mode=lax.GatherScatterMode.PROMISE_IN_BOUNDS)


def _sc_edge_pass(hs_hbm, ei_hbm, ef_hbm, out_hbm,
                  src_v, dst_v, ef_v, rows_v, invb_v, acc,
                  sem_i, sem_e, sem_g, sem_s):
    cid = lax.axis_index("c")
    sid = lax.axis_index("s")
    wid = cid * NS + sid

    def zrow(r, carry):
        for j in range(NJ):
            rows_v[0, r, pl.ds(LANES * j, LANES)] = jnp.zeros(
                (LANES,), jnp.float32)
        return carry

    lax.fori_loop(0, CH, zrow, 0)
    base0 = sid * ROWS_PT
    for off, sz in ZCHUNKS:
        pltpu.sync_copy(rows_v.at[0, pl.ds(0, sz)],
                        acc.at[pl.ds(base0 + off, sz)])

    @pl.when(sid == NS - 1)
    def _zero_tail():
        pltpu.sync_copy(rows_v.at[0, pl.ds(0, TAIL)],
                        acc.at[pl.ds(NS * ROWS_PT, TAIL)])

    plsc.subcore_barrier()

    lane = lax.iota(jnp.int32, LANES)
    bfly = [lane ^ (1 << k) for k in range(4)]

    def _chunk_base(i):
        return (wid + i * NW) * CH

    def _idx_start(i, s):
        base = _chunk_base(i)
        pltpu.async_copy(ei_hbm.at[pl.ds(base, CH)], src_v.at[s], sem_i)
        pltpu.async_copy(ei_hbm.at[pl.ds(E + base, CH)], dst_v.at[s], sem_i)

    def _idx_wait(i, s):
        base = _chunk_base(i)
        pltpu.make_async_copy(
            ei_hbm.at[pl.ds(base, CH)], src_v.at[s], sem_i).wait()
        pltpu.make_async_copy(
            ei_hbm.at[pl.ds(E + base, CH)], dst_v.at[s], sem_i).wait()

    _idx_start(0, 0)
    pltpu.async_copy(ef_hbm.at[pl.ds(_chunk_base(0), CH)], ef_v.at[0], sem_e)
    _idx_wait(0, 0)
    pltpu.async_copy(hs_hbm.at[src_v.at[0]], rows_v.at[0], sem_g)
    _idx_start(1, 1)

    def chunk_body(i, carry):
        b = jnp.bitwise_and(i, 1)
        nb = 1 - b
        s0 = lax.rem(i, 3)
        s1 = lax.rem(i + 1, 3)
        s2 = lax.rem(i + 2, 3)
        sp = lax.rem(i + 2, 3)
        pltpu.make_async_copy(
            hs_hbm.at[src_v.at[s0]], rows_v.at[b], sem_g).wait()
        pltpu.make_async_copy(
            ef_hbm.at[pl.ds(_chunk_base(i), CH)], ef_v.at[b], sem_e).wait()

        @pl.when(i < NCH - 1)
        def _start_next_gather():
            _idx_wait(i + 1, s1)
            pltpu.async_copy(hs_hbm.at[src_v.at[s1]], rows_v.at[nb], sem_g)

            @pl.when(i > 0)
            def _drain_prev_scatter():
                pltpu.make_async_copy(
                    ef_v.at[nb], acc.at[dst_v.at[sp]], sem_s).wait()

            pltpu.async_copy(
                ef_hbm.at[pl.ds(_chunk_base(i + 1), CH)], ef_v.at[nb], sem_e)

        @pl.when(i < NCH - 2)
        def _start_next_idx():
            _idx_start(i + 2, s2)

        @plsc.parallel_loop(0, CH, unroll=2)
        def rowsum(r):
            xs = []
            for j in range(NJ):
                x = jnp.exp(ef_v[b, r, pl.ds(LANES * j, LANES)])
                ef_v[b, r, pl.ds(LANES * j, LANES)] = x
                xs.append(x)
            t = [xs[2 * j] + xs[2 * j + 1] for j in range(4)]
            u = [t[0] + t[1], t[2] + t[3]]
            s = u[0] + u[1]
            for p in bfly:
                s = s + _lane_shuffle(s, p)
            invb_v[b, pl.ds(pl.multiple_of(r * LANES, 8), LANES)] = 1.0 / s

        @plsc.parallel_loop(0, CH, unroll=4)
        def rowmul(r):
            inv = invb_v[b, pl.ds(pl.multiple_of(r * LANES, 8), LANES)]
            for j in range(NJ):
                ef_v[b, r, pl.ds(LANES * j, LANES)] = (
                    ef_v[b, r, pl.ds(LANES * j, LANES)]
                    * (rows_v[b, r, pl.ds(LANES * j, LANES)] * inv))

        pltpu.async_copy(ef_v.at[b], acc.at[dst_v.at[s0]], sem_s, add=True)
        return carry

    lax.fori_loop(0, NCH, chunk_body, 0)

    lastb = jnp.int32((NCH - 1) % 2)
    pltpu.make_async_copy(
        ef_v.at[1 - lastb], acc.at[dst_v.at[jnp.int32((NCH - 2) % 3)]],
        sem_s).wait()
    pltpu.make_async_copy(
        ef_v.at[lastb], acc.at[dst_v.at[jnp.int32((NCH - 1) % 3)]],
        sem_s).wait()
    plsc.subcore_barrier()

    for off, sz in ZCHUNKS:
        start = base0 + off
        pltpu.sync_copy(acc.at[pl.ds(start, sz)], rows_v.at[0, pl.ds(0, sz)])
        pltpu.sync_copy(rows_v.at[0, pl.ds(0, sz)],
                        out_hbm.at[cid, pl.ds(start, sz)])

    @pl.when(sid == NS - 1)
    def _copy_tail():
        pltpu.sync_copy(acc.at[pl.ds(NS * ROWS_PT, TAIL)],
                        rows_v.at[0, pl.ds(0, TAIL)])
        pltpu.sync_copy(rows_v.at[0, pl.ds(0, TAIL)],
                        out_hbm.at[cid, pl.ds(NS * ROWS_PT, TAIL)])


_sc_call = functools.partial(
    pl.kernel,
    mesh=plsc.VectorSubcoreMesh(core_axis_name="c", subcore_axis_name="s"),
    out_type=jax.ShapeDtypeStruct((NC, N, D), jnp.float32),
    scratch_types=[
        pltpu.VMEM((3, CH), jnp.int32),
        pltpu.VMEM((3, CH), jnp.int32),
        pltpu.VMEM((2, CH, D), jnp.float32),
        pltpu.VMEM((2, CH, D), jnp.float32),
        pltpu.VMEM((2, CH * LANES), jnp.float32),
        pltpu.VMEM_SHARED((N, D), jnp.float32),
        pltpu.SemaphoreType.DMA,
        pltpu.SemaphoreType.DMA,
        pltpu.SemaphoreType.DMA,
        pltpu.SemaphoreType.DMA,
    ],
)(_sc_edge_pass)


def kernel(h, edge_index, edge_feat, weight, bias):
    hs = pl.pallas_call(
        _mm_body,
        grid=(5,),
        in_specs=[
            pl.BlockSpec((2000, D), lambda i: (i, 0)),
            pl.BlockSpec((2, D, D), lambda i: (0, 0, 0)),
        ],
        out_specs=pl.BlockSpec((2000, D), lambda i: (i, 0)),
        out_shape=jax.ShapeDtypeStruct((N, D), jnp.float32),
    )(h, weight)

    partials = _sc_call(hs, edge_index.reshape(2 * E), edge_feat)

    out = pl.pallas_call(
        _combine_body,
        grid=(5,),
        in_specs=[
            pl.BlockSpec((NC, 2000, D), lambda i: (0, i, 0)),
            pl.BlockSpec((1, D), lambda i: (0, 0)),
        ],
        out_specs=pl.BlockSpec((2000, D), lambda i: (i, 0)),
        out_shape=jax.ShapeDtypeStruct((N, D), jnp.float32),
    )(partials, bias.reshape(1, D))
    return out
